# baseline (speedup 1.0000x reference)
import functools

import jax
import jax.numpy as jnp
from jax import lax
from jax.experimental import pallas as pl
from jax.experimental.pallas import tpu as pltpu
from jax.experimental.pallas import tpu_sc as plsc

_BATCH = 16384
_NCAM = 100000
_ROW = 6
_OUT_ROWS = 12
_LANES = 16
_NC = 2
_NS = 16
_NW = _NC * _NS
_BPW = _BATCH // _NW
_ICHUNK = 128
_NCHUNK = _BPW // _ICHUNK
_NTILE = _BATCH // _ICHUNK

_F1 = (1.0, -1.0 / 6, 1.0 / 120, -1.0 / 5040, 1.0 / 362880, -1.0 / 39916800)
_F2 = (0.5, -1.0 / 24, 1.0 / 720, -1.0 / 40320, 1.0 / 3628800,
       -1.0 / 479001600)


def _poly(t, coeffs):
    acc = jnp.full((_LANES,), coeffs[-1], jnp.float32)
    for c in coeffs[-2::-1]:
        acc = acc * t + c
    return acc


def _sc_body(idx_hbm, tab_hbm, out_hbm, idx_v, idx6_v, comp_v, out_v,
             gsem, osem):
    wid = lax.axis_index("s") * _NC + lax.axis_index("c")
    base = wid * _BPW

    pltpu.sync_copy(idx_hbm.at[pl.ds(base, _BPW)], idx_v)

    def fill(i):
        v = idx_v[pl.ds(i * _LANES, _LANES)]
        for c in range(_ROW):
            idx6_v[c, pl.ds(i * _LANES, _LANES)] = v + c * _NCAM

    gathers = []
    for j in range(_NCHUNK):
        for i in range(j * (_ICHUNK // _LANES), (j + 1) * (_ICHUNK // _LANES)):
            fill(i)
        gathers.append([
            pltpu.async_copy(
                tab_hbm.at[idx6_v.at[c, pl.ds(j * _ICHUNK, _ICHUNK)]],
                comp_v.at[c, pl.ds(j * _ICHUNK, _ICHUNK)], gsem)
            for c in range(_ROW)
        ])

    def step(i):
        s = pl.ds(i * _LANES, _LANES)
        tx, ty, tz = comp_v[0, s], comp_v[1, s], comp_v[2, s]
        wx, wy, wz = comp_v[3, s], comp_v[4, s], comp_v[5, s]
        xx, yy, zz = wx * wx, wy * wy, wz * wz
        t = jnp.maximum(xx + yy + zz, 1e-8)
        f1 = _poly(t, _F1)
        f2 = _poly(t, _F2)
        xy, xz, yz = wx * wy, wx * wz, wy * wz
        f2xy, f2xz, f2yz = f2 * xy, f2 * xz, f2 * yz
        f1x, f1y, f1z = f1 * wx, f1 * wy, f1 * wz
        vals = (
            1.0 - f2 * (yy + zz), f2xy - f1z, f2xz + f1y, tx,
            f2xy + f1z, 1.0 - f2 * (xx + zz), f2yz - f1x, ty,
            f2xz - f1y, f2yz + f1x, 1.0 - f2 * (xx + yy), tz,
        )
        for r, v in enumerate(vals):
            out_v[r, s] = v

    outs = []
    for j in range(_NCHUNK):
        for cp in gathers[j]:
            cp.wait()
        for i in range(j * (_ICHUNK // _LANES), (j + 1) * (_ICHUNK // _LANES)):
            step(i)
        jg = wid * _NCHUNK + j
        outs.extend(
            pltpu.async_copy(out_v.at[r * 4 + c, pl.ds(j * _ICHUNK, _ICHUNK)],
                             out_hbm.at[r, jg, c], osem)
            for r in range(3) for c in range(4)
        )
    for cp in outs:
        cp.wait()


_sc_call = functools.partial(
    pl.kernel,
    mesh=plsc.VectorSubcoreMesh(core_axis_name="c", subcore_axis_name="s"),
    out_type=jax.ShapeDtypeStruct((3, _NTILE, 4, _ICHUNK), jnp.float32),
    scratch_types=[
        pltpu.VMEM((_BPW,), jnp.int32),
        pltpu.VMEM((_ROW, _BPW), jnp.int32),
        pltpu.VMEM((_ROW, _BPW), jnp.float32),
        pltpu.VMEM((_OUT_ROWS, _BPW), jnp.float32),
        pltpu.SemaphoreType.DMA,
        pltpu.SemaphoreType.DMA,
    ],
    compiler_params=pltpu.CompilerParams(
        needs_layout_passes=False, use_tc_tiling_on_sc=False),
)(_sc_body)


def kernel(indices, pose_adjustment):
    idx = indices.astype(jnp.int32)
    tab = pose_adjustment.T.reshape(_ROW * _NCAM)
    out = _sc_call(idx, tab)
    return out.transpose(1, 3, 0, 2).reshape(_BATCH, 3, 4)


# speedup vs baseline: 1.0378x; 1.0378x over previous
import functools

import jax
import jax.numpy as jnp
from jax import lax
from jax.experimental import pallas as pl
from jax.experimental.pallas import tpu as pltpu
from jax.experimental.pallas import tpu_sc as plsc

_BATCH = 16384
_NCAM = 100000
_ROW = 6
_OUT_ROWS = 12
_LANES = 16
_NC = 2
_NS = 16
_NW = _NC * _NS
_BPW = _BATCH // _NW
_ICHUNK = 128
_NCHUNK = _BPW // _ICHUNK
_NTILE = _BATCH // _ICHUNK

_F1 = (1.0, -1.0 / 6, 1.0 / 120, -1.0 / 5040)
_F2 = (0.5, -1.0 / 24, 1.0 / 720, -1.0 / 40320)


def _poly(t, coeffs):
    acc = jnp.full((_LANES,), coeffs[-1], jnp.float32)
    for c in coeffs[-2::-1]:
        acc = acc * t + c
    return acc


def _sc_body(idx_hbm, tab_hbm, out_hbm, idx_v, idx6_v, comp_v, out_v,
             gsem, osem):
    wid = lax.axis_index("s") * _NC + lax.axis_index("c")
    base = wid * _BPW

    pltpu.sync_copy(idx_hbm.at[pl.ds(base, _BPW)], idx_v)

    def fill(i, carry):
        v = idx_v[pl.ds(i * _LANES, _LANES)]
        for c in range(_ROW):
            idx6_v[c, pl.ds(i * _LANES, _LANES)] = v + c * _NCAM
        return carry

    gathers = []
    for j in range(_NCHUNK):
        lax.fori_loop(j * (_ICHUNK // _LANES), (j + 1) * (_ICHUNK // _LANES),
                      fill, 0)
        gathers.append([
            pltpu.async_copy(
                tab_hbm.at[idx6_v.at[c, pl.ds(j * _ICHUNK, _ICHUNK)]],
                comp_v.at[c, pl.ds(j * _ICHUNK, _ICHUNK)], gsem)
            for c in range(_ROW)
        ])

    def make_step(j):
        def step(i, carry):
            s = pl.ds(i * _LANES, _LANES)
            sl = pl.ds(i * _LANES - j * _ICHUNK, _LANES)
            tx, ty, tz = comp_v[0, s], comp_v[1, s], comp_v[2, s]
            wx, wy, wz = comp_v[3, s], comp_v[4, s], comp_v[5, s]
            xx, yy, zz = wx * wx, wy * wy, wz * wz
            t = jnp.maximum(xx + yy + zz, 1e-8)
            f1 = _poly(t, _F1)
            f2 = _poly(t, _F2)
            xy, xz, yz = wx * wy, wx * wz, wy * wz
            f2xy, f2xz, f2yz = f2 * xy, f2 * xz, f2 * yz
            f1x, f1y, f1z = f1 * wx, f1 * wy, f1 * wz
            vals = (
                1.0 - f2 * (yy + zz), f2xy - f1z, f2xz + f1y, tx,
                f2xy + f1z, 1.0 - f2 * (xx + zz), f2yz - f1x, ty,
                f2xz - f1y, f2yz + f1x, 1.0 - f2 * (xx + yy), tz,
            )
            for r3 in range(3):
                for c4 in range(4):
                    out_v[r3, j, c4, sl] = vals[r3 * 4 + c4]
            return carry
        return step

    for j in range(_NCHUNK):
        for cp in gathers[j]:
            cp.wait()
        lax.fori_loop(j * (_ICHUNK // _LANES), (j + 1) * (_ICHUNK // _LANES),
                      make_step(j), 0)
    jg0 = wid * _NCHUNK
    outs = [
        pltpu.async_copy(out_v.at[r], out_hbm.at[r, pl.ds(jg0, _NCHUNK)], osem)
        for r in range(3)
    ]
    for cp in outs:
        cp.wait()


_sc_call = functools.partial(
    pl.kernel,
    mesh=plsc.VectorSubcoreMesh(core_axis_name="c", subcore_axis_name="s"),
    out_type=jax.ShapeDtypeStruct((3, _NTILE, 4, _ICHUNK), jnp.float32),
    scratch_types=[
        pltpu.VMEM((_BPW,), jnp.int32),
        pltpu.VMEM((_ROW, _BPW), jnp.int32),
        pltpu.VMEM((_ROW, _BPW), jnp.float32),
        pltpu.VMEM((3, _NCHUNK, 4, _ICHUNK), jnp.float32),
        pltpu.SemaphoreType.DMA,
        pltpu.SemaphoreType.DMA,
    ],
    compiler_params=pltpu.CompilerParams(
        needs_layout_passes=False, use_tc_tiling_on_sc=False),
)(_sc_body)


def kernel(indices, pose_adjustment):
    idx = indices.astype(jnp.int32)
    tab = pose_adjustment.T.reshape(_ROW * _NCAM)
    out = _sc_call(idx, tab)
    return out.transpose(1, 3, 0, 2).reshape(_BATCH, 3, 4)


# speedup vs baseline: 1.0456x; 1.0076x over previous
import functools

import jax
import jax.numpy as jnp
from jax import lax
from jax.experimental import pallas as pl
from jax.experimental.pallas import tpu as pltpu
from jax.experimental.pallas import tpu_sc as plsc

_BATCH = 16384
_NCAM = 100000
_ROW = 6
_OUT_ROWS = 12
_LANES = 16
_NC = 2
_NS = 16
_NW = _NC * _NS
_BPW = _BATCH // _NW
_ICHUNK = 128
_NCHUNK = _BPW // _ICHUNK
_NTILE = _BATCH // _ICHUNK

_F1 = (1.0, -1.0 / 6, 1.0 / 120, -1.0 / 5040)
_F2 = (0.5, -1.0 / 24, 1.0 / 720, -1.0 / 40320)


def _poly(t, coeffs):
    acc = jnp.full((_LANES,), coeffs[-1], jnp.float32)
    for c in coeffs[-2::-1]:
        acc = acc * t + c
    return acc


def _sc_body(idx_hbm, tab_hbm, out_hbm, idx_v, comp_v, out_v, gsem, osem):
    wid = lax.axis_index("s") * _NC + lax.axis_index("c")
    base = wid * _BPW

    pltpu.sync_copy(idx_hbm.at[pl.ds(base, _BPW)], idx_v)

    gathers = []
    for j in range(_NCHUNK):
        gathers.append([
            pltpu.async_copy(
                tab_hbm.at[pl.ds(c * _NCAM, _NCAM)]
                       .at[idx_v.at[pl.ds(j * _ICHUNK, _ICHUNK)]],
                comp_v.at[c, pl.ds(j * _ICHUNK, _ICHUNK)], gsem)
            for c in range(_ROW)
        ])

    def make_step(j):
        def step(i, carry):
            s = pl.ds(i * _LANES, _LANES)
            sl = pl.ds(i * _LANES - j * _ICHUNK, _LANES)
            tx, ty, tz = comp_v[0, s], comp_v[1, s], comp_v[2, s]
            wx, wy, wz = comp_v[3, s], comp_v[4, s], comp_v[5, s]
            xx, yy, zz = wx * wx, wy * wy, wz * wz
            t = jnp.maximum(xx + yy + zz, 1e-8)
            f1 = _poly(t, _F1)
            f2 = _poly(t, _F2)
            xy, xz, yz = wx * wy, wx * wz, wy * wz
            f2xy, f2xz, f2yz = f2 * xy, f2 * xz, f2 * yz
            f1x, f1y, f1z = f1 * wx, f1 * wy, f1 * wz
            vals = (
                1.0 - f2 * (yy + zz), f2xy - f1z, f2xz + f1y, tx,
                f2xy + f1z, 1.0 - f2 * (xx + zz), f2yz - f1x, ty,
                f2xz - f1y, f2yz + f1x, 1.0 - f2 * (xx + yy), tz,
            )
            for r3 in range(3):
                for c4 in range(4):
                    out_v[r3, j, c4, sl] = vals[r3 * 4 + c4]
            return carry
        return step

    for j in range(_NCHUNK):
        for cp in gathers[j]:
            cp.wait()
        lax.fori_loop(j * (_ICHUNK // _LANES), (j + 1) * (_ICHUNK // _LANES),
                      make_step(j), 0)
    jg0 = wid * _NCHUNK
    outs = [
        pltpu.async_copy(out_v.at[r], out_hbm.at[r, pl.ds(jg0, _NCHUNK)], osem)
        for r in range(3)
    ]
    for cp in outs:
        cp.wait()


_sc_call = functools.partial(
    pl.kernel,
    mesh=plsc.VectorSubcoreMesh(core_axis_name="c", subcore_axis_name="s"),
    out_type=jax.ShapeDtypeStruct((3, _NTILE, 4, _ICHUNK), jnp.float32),
    scratch_types=[
        pltpu.VMEM((_BPW,), jnp.int32),
        pltpu.VMEM((_ROW, _BPW), jnp.float32),
        pltpu.VMEM((3, _NCHUNK, 4, _ICHUNK), jnp.float32),
        pltpu.SemaphoreType.DMA,
        pltpu.SemaphoreType.DMA,
    ],
    compiler_params=pltpu.CompilerParams(
        needs_layout_passes=False, use_tc_tiling_on_sc=False),
)(_sc_body)


def kernel(indices, pose_adjustment):
    idx = indices.astype(jnp.int32)
    tab = pose_adjustment.T.reshape(_ROW * _NCAM)
    out = _sc_call(idx, tab)
    return out.transpose(1, 3, 0, 2).reshape(_BATCH, 3, 4)
